# Initial kernel scaffold; baseline (speedup 1.0000x reference)
import functools

import jax
import jax.numpy as jnp
from jax import lax
from jax.experimental import pallas as pl
from jax.experimental.pallas import tpu as pltpu
from jax.experimental.pallas import tpu_sc as plsc

N_NODES = 10000
D = 128
E = 320000

NC = 2
NS = 16
NW = NC * NS
CHUNK = 128
NCHUNKS = 79
E_PAD = NW * NCHUNKS * CHUNK
N_PAD = 10240
RPT = N_PAD // NS

_MESH = plsc.VectorSubcoreMesh(core_axis_name="c", subcore_axis_name="s")


def _sc_degree(dst_t, ones_chunk, zeros_deg):
    @functools.partial(
        pl.kernel,
        out_type=jax.ShapeDtypeStruct((NC * N_PAD, 16), jnp.float32),
        mesh=_MESH,
        scratch_types=[
            pltpu.VMEM_SHARED((N_PAD, 16), jnp.float32),
            pltpu.VMEM((NCHUNKS, CHUNK), jnp.int32),
            pltpu.VMEM((CHUNK, 16), jnp.float32),
        ],
    )
    def k(dst_hbm, ones_hbm, zeros_hbm, out_hbm, acc, idx_v, ones_v):
        cid = lax.axis_index("c")
        sid = lax.axis_index("s")
        wid = sid * NC + cid
        pltpu.sync_copy(zeros_hbm.at[pl.ds(sid * RPT, RPT)],
                        acc.at[pl.ds(sid * RPT, RPT)])
        pltpu.sync_copy(dst_hbm.at[wid], idx_v)
        pltpu.sync_copy(ones_hbm, ones_v)
        plsc.subcore_barrier()

        @pl.loop(0, NCHUNKS)
        def _(j):
            pltpu.sync_copy(ones_v, acc.at[idx_v.at[j]], add=True)

        plsc.subcore_barrier()
        pltpu.sync_copy(acc.at[pl.ds(sid * RPT, RPT)],
                        out_hbm.at[pl.ds(cid * N_PAD + sid * RPT, RPT)])

    return k(dst_t, ones_chunk, zeros_deg)


def _sc_segsum(h, src_t, dst_t, zeros_big):
    @functools.partial(
        pl.kernel,
        out_type=jax.ShapeDtypeStruct((NC * N_PAD, D), jnp.float32),
        mesh=_MESH,
        scratch_types=[
            pltpu.VMEM_SHARED((N_PAD, D), jnp.float32),
            pltpu.VMEM((NCHUNKS, CHUNK), jnp.int32),
            pltpu.VMEM((NCHUNKS, CHUNK), jnp.int32),
            pltpu.VMEM((CHUNK, D), jnp.float32),
            pltpu.VMEM((CHUNK, D), jnp.float32),
            pltpu.SemaphoreType.DMA,
            pltpu.SemaphoreType.DMA,
        ],
    )
    def k(h_hbm, src_hbm, dst_hbm, zeros_hbm, out_hbm,
          acc, srcv, dstv, buf0, buf1, sem0, sem1):
        cid = lax.axis_index("c")
        sid = lax.axis_index("s")
        wid = sid * NC + cid
        pltpu.sync_copy(zeros_hbm.at[pl.ds(sid * RPT, RPT)],
                        acc.at[pl.ds(sid * RPT, RPT)])
        pltpu.sync_copy(src_hbm.at[wid], srcv)
        pltpu.sync_copy(dst_hbm.at[wid], dstv)
        plsc.subcore_barrier()

        pltpu.async_copy(h_hbm.at[srcv.at[0]], buf0, sem0)

        @pl.loop(0, NCHUNKS, step=2)
        def _(j):
            pltpu.make_async_copy(h_hbm.at[srcv.at[j]], buf0, sem0).wait()

            @pl.when(j + 1 < NCHUNKS)
            def _():
                pltpu.async_copy(h_hbm.at[srcv.at[j + 1]], buf1, sem1)

            pltpu.sync_copy(buf0, acc.at[dstv.at[j]], add=True)

            @pl.when(j + 1 < NCHUNKS)
            def _():
                pltpu.make_async_copy(h_hbm.at[srcv.at[j + 1]], buf1, sem1).wait()

                @pl.when(j + 2 < NCHUNKS)
                def _():
                    pltpu.async_copy(h_hbm.at[srcv.at[j + 2]], buf0, sem0)

                pltpu.sync_copy(buf1, acc.at[dstv.at[j + 1]], add=True)

        plsc.subcore_barrier()
        pltpu.sync_copy(acc.at[pl.ds(sid * RPT, RPT)],
                        out_hbm.at[pl.ds(cid * N_PAD + sid * RPT, RPT)])

    return k(h, src_t, dst_t, zeros_big)


def _tc_matmul(x, W):
    def body(x_ref, w_ref, o_ref):
        o_ref[...] = lax.dot_general(
            x_ref[...], w_ref[...], (((1,), (0,)), ((), ())),
            precision=lax.Precision.HIGHEST,
            preferred_element_type=jnp.float32)

    return pl.pallas_call(
        body,
        out_shape=jax.ShapeDtypeStruct((N_NODES, D), jnp.float32),
    )(x, W)


def _tc_scale(deg_p, h1):
    def body(deg_ref, h_ref, h1p_ref, dis_ref):
        deg = (deg_ref[0:N_NODES, 0:1]
               + deg_ref[N_PAD:N_PAD + N_NODES, 0:1] + 1.0)
        dis = lax.rsqrt(deg)
        dis_ref[...] = dis
        h1p_ref[...] = h_ref[...] * dis

    return pl.pallas_call(
        body,
        out_shape=(jax.ShapeDtypeStruct((N_NODES, D), jnp.float32),
                   jax.ShapeDtypeStruct((N_NODES, 1), jnp.float32)),
    )(deg_p, h1)


def _tc_mid(s1, h1p, dis, b1, W2):
    def body(s_ref, h1p_ref, dis_ref, b1_ref, w2_ref, o_ref):
        s = (s_ref[0:N_NODES, :] + s_ref[N_PAD:N_PAD + N_NODES, :]
             + h1p_ref[...])
        z = dis_ref[...] * s + b1_ref[...][None, :]
        x2 = jnp.where(z >= 0, z, 0.01 * z)
        h2 = lax.dot_general(
            x2, w2_ref[...], (((1,), (0,)), ((), ())),
            precision=lax.Precision.HIGHEST,
            preferred_element_type=jnp.float32)
        o_ref[...] = h2 * dis_ref[...]

    return pl.pallas_call(
        body,
        out_shape=jax.ShapeDtypeStruct((N_NODES, D), jnp.float32),
    )(s1, h1p, dis, b1, W2)


def _tc_final(s2, h2p, dis, b2):
    def body(s_ref, h2p_ref, dis_ref, b2_ref, o_ref):
        s = (s_ref[0:N_NODES, :] + s_ref[N_PAD:N_PAD + N_NODES, :]
             + h2p_ref[...])
        o_ref[...] = dis_ref[...] * s + b2_ref[...][None, :]

    return pl.pallas_call(
        body,
        out_shape=jax.ShapeDtypeStruct((N_NODES, D), jnp.float32),
    )(s2, h2p, dis, b2)


def kernel(x, edge_index, W1, b1, W2, b2):
    src = edge_index[0].astype(jnp.int32)
    dst = edge_index[1].astype(jnp.int32)

    npad = E_PAD - E
    pad_ar = jnp.arange(npad, dtype=jnp.int32)
    pad_src = pad_ar % N_NODES
    pad_dst = N_NODES + pad_ar % (N_PAD - N_NODES)
    src_t = jnp.concatenate([src, pad_src]).reshape(NW, NCHUNKS, CHUNK)
    dst_t = jnp.concatenate([dst, pad_dst]).reshape(NW, NCHUNKS, CHUNK)

    ones_chunk = jnp.ones((CHUNK, 16), jnp.float32)
    zeros_deg = jnp.zeros((N_PAD, 16), jnp.float32)
    zeros_big = jnp.zeros((N_PAD, D), jnp.float32)

    deg_p = _sc_degree(dst_t, ones_chunk, zeros_deg)
    h1 = _tc_matmul(x, W1)
    h1p, dis = _tc_scale(deg_p, h1)
    s1 = _sc_segsum(h1p, src_t, dst_t, zeros_big)
    h2p = _tc_mid(s1, h1p, dis, b1, W2)
    s2 = _sc_segsum(h2p, src_t, dst_t, zeros_big)
    return _tc_final(s2, h2p, dis, b2)


# speedup vs baseline: 25.1276x; 25.1276x over previous
import functools

import jax
import jax.numpy as jnp
from jax import lax
from jax.experimental import pallas as pl
from jax.experimental.pallas import tpu as pltpu
from jax.experimental.pallas import tpu_sc as plsc

N_NODES = 10000
D = 128
E = 320000

NC = 2
NS = 16
NW = NC * NS
CHUNK = 128
NCHUNKS = 79
E_PAD = NW * NCHUNKS * CHUNK
N_PAD = 10112
RPT = N_PAD // NS

_MESH = plsc.VectorSubcoreMesh(core_axis_name="c", subcore_axis_name="s")


def _sc_degree(dst_t, ones_chunk, zeros_big):
    @functools.partial(
        pl.kernel,
        out_type=jax.ShapeDtypeStruct((NC * N_PAD, D), jnp.float32),
        mesh=_MESH,
        scratch_types=[
            pltpu.VMEM_SHARED((N_PAD, D), jnp.float32),
            pltpu.VMEM((NCHUNKS, CHUNK), jnp.int32),
            pltpu.VMEM((CHUNK, D), jnp.float32),
            pltpu.SemaphoreType.DMA,
        ],
    )
    def k(dst_hbm, ones_hbm, zeros_hbm, out_hbm, acc, idx_v, ones_v, sem):
        cid = lax.axis_index("c")
        sid = lax.axis_index("s")
        wid = sid * NC + cid
        pltpu.sync_copy(zeros_hbm.at[pl.ds(sid * RPT, RPT)],
                        acc.at[pl.ds(sid * RPT, RPT)])
        pltpu.sync_copy(dst_hbm.at[wid], idx_v)
        pltpu.sync_copy(ones_hbm, ones_v)
        plsc.subcore_barrier()

        @pl.loop(0, NCHUNKS)
        def _(j):
            pltpu.async_copy(ones_v, acc.at[idx_v.at[j]], sem, add=True)

        @pl.loop(0, NCHUNKS)
        def _(j):
            pltpu.make_async_copy(ones_v, acc.at[idx_v.at[j]], sem).wait()

        plsc.subcore_barrier()
        pltpu.sync_copy(acc.at[pl.ds(sid * RPT, RPT)],
                        out_hbm.at[pl.ds(cid * N_PAD + sid * RPT, RPT)])

    return k(dst_t, ones_chunk, zeros_big)


def _sc_segsum(h, idx_t, zeros_big):
    @functools.partial(
        pl.kernel,
        out_type=jax.ShapeDtypeStruct((NC * N_PAD, D), jnp.float32),
        mesh=_MESH,
        scratch_types=[
            pltpu.VMEM_SHARED((N_PAD, D), jnp.float32),
            pltpu.VMEM((2, CHUNK), jnp.int32),
            pltpu.VMEM((2, CHUNK), jnp.int32),
            pltpu.VMEM((CHUNK, D), jnp.float32),
            pltpu.VMEM((CHUNK, D), jnp.float32),
            pltpu.SemaphoreType.DMA,
            pltpu.SemaphoreType.DMA,
            pltpu.SemaphoreType.DMA,
            pltpu.SemaphoreType.DMA,
        ],
    )
    def k(h_hbm, idx_hbm, zeros_hbm, out_hbm,
          acc, ring0, ring1, buf0, buf1, gsem0, gsem1, isem0, isem1):
        cid = lax.axis_index("c")
        sid = lax.axis_index("s")
        wid = sid * NC + cid
        pltpu.sync_copy(zeros_hbm.at[pl.ds(sid * RPT, RPT)],
                        acc.at[pl.ds(sid * RPT, RPT)])
        pltpu.sync_copy(idx_hbm.at[wid, 0], ring0)
        plsc.subcore_barrier()

        pltpu.async_copy(h_hbm.at[ring0.at[0]], buf0, gsem0)
        pltpu.async_copy(idx_hbm.at[wid, 1], ring1, isem1)

        @pl.loop(0, NCHUNKS, step=2)
        def _(j):
            pltpu.make_async_copy(h_hbm.at[ring0.at[0]], buf0, gsem0).wait()

            @pl.when(j + 1 < NCHUNKS)
            def _():
                pltpu.make_async_copy(idx_hbm.at[wid, j + 1], ring1,
                                      isem1).wait()
                pltpu.async_copy(h_hbm.at[ring1.at[0]], buf1, gsem1)

            pltpu.sync_copy(buf0, acc.at[ring0.at[1]], add=True)

            @pl.when(j + 2 < NCHUNKS)
            def _():
                pltpu.async_copy(idx_hbm.at[wid, j + 2], ring0, isem0)

            @pl.when(j + 1 < NCHUNKS)
            def _():
                pltpu.make_async_copy(h_hbm.at[ring1.at[0]], buf1,
                                      gsem1).wait()

                @pl.when(j + 2 < NCHUNKS)
                def _():
                    pltpu.make_async_copy(idx_hbm.at[wid, j + 2], ring0,
                                          isem0).wait()
                    pltpu.async_copy(h_hbm.at[ring0.at[0]], buf0, gsem0)

                pltpu.sync_copy(buf1, acc.at[ring1.at[1]], add=True)

                @pl.when(j + 3 < NCHUNKS)
                def _():
                    pltpu.async_copy(idx_hbm.at[wid, j + 3], ring1, isem1)

        plsc.subcore_barrier()
        pltpu.sync_copy(acc.at[pl.ds(sid * RPT, RPT)],
                        out_hbm.at[pl.ds(cid * N_PAD + sid * RPT, RPT)])

    return k(h, idx_t, zeros_big)


def _tc_matmul(x, W):
    def body(x_ref, w_ref, o_ref):
        o_ref[...] = lax.dot_general(
            x_ref[...], w_ref[...], (((1,), (0,)), ((), ())),
            precision=lax.Precision.HIGHEST,
            preferred_element_type=jnp.float32)

    return pl.pallas_call(
        body,
        out_shape=jax.ShapeDtypeStruct((N_NODES, D), jnp.float32),
    )(x, W)


def _tc_scale(deg_p, h1):
    def body(deg_ref, h_ref, h1p_ref, dis_ref):
        deg = (deg_ref[0:N_NODES, 0:1]
               + deg_ref[N_PAD:N_PAD + N_NODES, 0:1] + 1.0)
        dis = lax.rsqrt(deg)
        dis_ref[...] = dis
        h1p_ref[...] = h_ref[...] * dis

    return pl.pallas_call(
        body,
        out_shape=(jax.ShapeDtypeStruct((N_NODES, D), jnp.float32),
                   jax.ShapeDtypeStruct((N_NODES, 1), jnp.float32)),
    )(deg_p, h1)


def _tc_mid(s1, h1p, dis, b1, W2):
    def body(s_ref, h1p_ref, dis_ref, b1_ref, w2_ref, o_ref):
        s = (s_ref[0:N_NODES, :] + s_ref[N_PAD:N_PAD + N_NODES, :]
             + h1p_ref[...])
        z = dis_ref[...] * s + b1_ref[...][None, :]
        x2 = jnp.where(z >= 0, z, 0.01 * z)
        h2 = lax.dot_general(
            x2, w2_ref[...], (((1,), (0,)), ((), ())),
            precision=lax.Precision.HIGHEST,
            preferred_element_type=jnp.float32)
        o_ref[...] = h2 * dis_ref[...]

    return pl.pallas_call(
        body,
        out_shape=jax.ShapeDtypeStruct((N_NODES, D), jnp.float32),
    )(s1, h1p, dis, b1, W2)


def _tc_final(s2, h2p, dis, b2):
    def body(s_ref, h2p_ref, dis_ref, b2_ref, o_ref):
        s = (s_ref[0:N_NODES, :] + s_ref[N_PAD:N_PAD + N_NODES, :]
             + h2p_ref[...])
        o_ref[...] = dis_ref[...] * s + b2_ref[...][None, :]

    return pl.pallas_call(
        body,
        out_shape=jax.ShapeDtypeStruct((N_NODES, D), jnp.float32),
    )(s2, h2p, dis, b2)


def kernel(x, edge_index, W1, b1, W2, b2):
    src = edge_index[0].astype(jnp.int32)
    dst = edge_index[1].astype(jnp.int32)

    npad = E_PAD - E
    pad_ar = jnp.arange(npad, dtype=jnp.int32)
    pad_src = pad_ar % N_NODES
    pad_dst = N_NODES + pad_ar % (N_PAD - N_NODES)
    src_t = jnp.concatenate([src, pad_src]).reshape(NW, NCHUNKS, CHUNK)
    dst_t = jnp.concatenate([dst, pad_dst]).reshape(NW, NCHUNKS, CHUNK)
    idx_t = jnp.stack([src_t, dst_t], axis=2)

    ones_chunk = jnp.ones((CHUNK, D), jnp.float32)
    zeros_big = jnp.zeros((N_PAD, D), jnp.float32)

    deg_p = _sc_degree(dst_t, ones_chunk, zeros_big)
    h1 = _tc_matmul(x, W1)
    h1p, dis = _tc_scale(deg_p, h1)
    s1 = _sc_segsum(h1p, idx_t, zeros_big)
    h2p = _tc_mid(s1, h1p, dis, b1, W2)
    s2 = _sc_segsum(h2p, idx_t, zeros_big)
    return _tc_final(s2, h2p, dis, b2)
